# baseline (speedup 1.0000x reference)
import functools

import jax
import jax.numpy as jnp
import numpy as np
from jax.experimental import pallas as pl
from jax.experimental.pallas import tpu as pltpu

VAR0 = 0.1
VAR1 = 0.2
SIGMA_REPGT = 0.9
EPS = 1e-10
LOG1MS = np.float32(np.log(1.0 - SIGMA_REPGT))

TN = 512


def _decode_cols(l4n, p4n):
    lx, ly, lw, lh = l4n[0:1, :], l4n[1:2, :], l4n[2:3, :], l4n[3:4, :]
    px, py, pw, ph = p4n[0:1, :], p4n[1:2, :], p4n[2:3, :], p4n[3:4, :]
    cx = px + lx * VAR0 * pw
    cy = py + ly * VAR0 * ph
    w = pw * jnp.exp(lw * VAR1)
    h = ph * jnp.exp(lh * VAR1)
    x1 = cx - w * 0.5
    y1 = cy - h * 0.5
    x2 = cx + w * 0.5
    y2 = cy + h * 0.5
    area = (x2 - x1) * (y2 - y1)
    return x1, y1, x2, y2, area


def _decode_rows(lr, pr):
    lx, ly, lw, lh = lr[:, 0:1], lr[:, 1:2], lr[:, 2:3], lr[:, 3:4]
    px, py, pw, ph = pr[:, 0:1], pr[:, 1:2], pr[:, 2:3], pr[:, 3:4]
    cx = px + lx * VAR0 * pw
    cy = py + ly * VAR0 * ph
    w = pw * jnp.exp(lw * VAR1)
    h = ph * jnp.exp(lh * VAR1)
    x1 = cx - w * 0.5
    y1 = cy - h * 0.5
    x2 = cx + w * 0.5
    y2 = cy + h * 0.5
    area = (x2 - x1) * (y2 - y1)
    return x1, y1, x2, y2, area


def _rep_kernel(n_tiles, n_sub, g, flags, loc_r, pri_r, loc_c, pri_c, m_col,
                gt_c, msub, out_ref, accb, accn, accg):
    bi = pl.program_id(0)
    t = pl.program_id(1)

    @pl.when(t == 0)
    def _():
        accb[...] = jnp.zeros_like(accb)
        accn[...] = jnp.zeros_like(accn)
        accg[...] = jnp.zeros_like(accg)

    x1c, y1c, x2c, y2c, area_c = _decode_cols(loc_c[0], pri_c[...])
    mc = m_col[0]

    gt = gt_c[0]
    gx1, gy1, gx2, gy2 = gt[0:1, :], gt[1:2, :], gt[2:3, :], gt[3:4, :]
    garea = (gx2 - gx1) * (gy2 - gy1)
    cols = jax.lax.broadcasted_iota(jnp.int32, (8, g), 1)

    msub_v = msub[0, 0]

    for j in range(n_sub):
        @pl.when(flags[bi, t, j] != 0)
        def _(j=j):
            x1r, y1r, x2r, y2r, area_r = _decode_rows(
                loc_r[0, j * 8:(j + 1) * 8, :], pri_r[j * 8:(j + 1) * 8, :])
            mr = msub_v[:, j:j + 1]

            pm = mr * mc
            iw = jnp.maximum(jnp.minimum(x2r, x2c) - jnp.maximum(x1r, x1c), 0.0)
            ih = jnp.maximum(jnp.minimum(y2r, y2c) - jnp.maximum(y1r, y1c), 0.0)
            inter = iw * ih
            ov = inter / (area_r + area_c - inter)
            vv = jnp.where(inter > 0.0, pm, 0.0)
            accb[...] += vv * ov
            accn[...] += vv

            giw = jnp.maximum(jnp.minimum(x2r, gx2) - jnp.maximum(x1r, gx1), 0.0)
            gih = jnp.maximum(jnp.minimum(y2r, gy2) - jnp.maximum(y1r, gy1), 0.0)
            ginter = giw * gih
            gov = (ginter / (area_r + garea - ginter)) * mr

            max1 = jnp.max(gov, axis=1, keepdims=True)
            arg1 = jnp.min(jnp.where(gov == max1, cols, g), axis=1,
                           keepdims=True)
            ov2 = jnp.where(cols == arg1, 0.0, gov)
            max2 = jnp.max(ov2, axis=1, keepdims=True)
            arg2 = jnp.min(jnp.where(ov2 == max2, cols, g), axis=1,
                           keepdims=True)
            onehot2 = jnp.where(cols == arg2, 1.0, 0.0)

            def sel(v):
                return jnp.sum(onehot2 * v, axis=1, keepdims=True)

            sx1, sy1, sx2, sy2 = sel(gx1), sel(gy1), sel(gx2), sel(gy2)
            sarea = sel(garea)
            iw2 = jnp.maximum(jnp.minimum(x2r, sx2) - jnp.maximum(x1r, sx1), 0.0)
            ih2 = jnp.maximum(jnp.minimum(y2r, sy2) - jnp.maximum(y1r, sy1), 0.0)
            iog = (iw2 * ih2) / sarea
            iog_safe = jnp.where(iog > SIGMA_REPGT, 0.0, iog)
            term = jnp.where(iog > SIGMA_REPGT,
                             (iog - SIGMA_REPGT) / (1.0 - SIGMA_REPGT) - LOG1MS,
                             -jnp.log(jnp.maximum(1.0 - iog_safe, EPS)))
            cf = jnp.where(max2 > 0.0, mr, 0.0)
            accg[:, 0:1] += cf * term
            accg[:, 1:2] += cf

    @pl.when(t == n_tiles - 1)
    def _():
        tb = jnp.sum(accb[...])
        nb = jnp.sum(accn[...])
        tg = jnp.sum(accg[:, 0:1])
        ng = jnp.sum(accg[:, 1:2])
        lgt = jnp.where(ng > 0.0, tg / jnp.maximum(ng, 1.0), 0.0)
        lbx = jnp.where(nb > 0.0, tb / jnp.maximum(nb, 1.0), 0.0)
        lane = jax.lax.broadcasted_iota(jnp.int32, (1, 1, 128), 2)
        out_ref[...] = jnp.where(lane == 0, lgt + lbx, 0.0)


@jax.jit
def kernel(loc_data, ground_data, prior_data, pos_idx):
    b, n, _ = loc_data.shape
    g = ground_data.shape[1]
    n_tiles = n // TN
    n_sub = TN // 8

    mask = pos_idx[..., 0]
    maskf = mask.astype(jnp.float32)
    loc_col = jnp.transpose(loc_data, (0, 2, 1))
    prior_col = prior_data.T
    gt_col = jnp.transpose(ground_data, (0, 2, 1))
    mask_col = maskf.reshape(b, 1, n)
    m4 = mask.reshape(b, n_tiles, n_sub, 8)
    mask_sub = m4.transpose(0, 1, 3, 2).astype(jnp.float32)
    flags = jnp.any(m4, axis=-1).astype(jnp.int32)

    out = pl.pallas_call(
        functools.partial(_rep_kernel, n_tiles, n_sub, g),
        out_shape=jax.ShapeDtypeStruct((b, 1, 128), jnp.float32),
        grid=(b, n_tiles),
        in_specs=[
            pl.BlockSpec(memory_space=pltpu.SMEM),
            pl.BlockSpec((1, TN, 4), lambda i, j: (i, j, 0)),
            pl.BlockSpec((TN, 4), lambda i, j: (j, 0)),
            pl.BlockSpec((1, 4, n), lambda i, j: (i, 0, 0)),
            pl.BlockSpec((4, n), lambda i, j: (0, 0)),
            pl.BlockSpec((1, 1, n), lambda i, j: (i, 0, 0)),
            pl.BlockSpec((1, 4, g), lambda i, j: (i, 0, 0)),
            pl.BlockSpec((1, 1, 8, n_sub), lambda i, j: (i, j, 0, 0)),
        ],
        out_specs=pl.BlockSpec((1, 1, 128), lambda i, j: (i, 0, 0)),
        scratch_shapes=[
            pltpu.VMEM((8, n), jnp.float32),
            pltpu.VMEM((8, n), jnp.float32),
            pltpu.VMEM((8, 128), jnp.float32),
        ],
        compiler_params=pltpu.CompilerParams(
            dimension_semantics=("parallel", "arbitrary"),
        ),
        name="repulsion_loss",
    )(flags, loc_data, prior_data, loc_col, prior_col, mask_col, gt_col,
      mask_sub)

    return jnp.sum(out[:, 0, 0])


# speedup vs baseline: 1.1391x; 1.1391x over previous
import functools

import jax
import jax.numpy as jnp
import numpy as np
from jax.experimental import pallas as pl
from jax.experimental.pallas import tpu as pltpu

VAR0 = 0.1
VAR1 = 0.2
SIGMA_REPGT = 0.9
EPS = 1e-10
LOG1MS = np.float32(np.log(1.0 - SIGMA_REPGT))
NEG = np.float32(-np.inf)


def _decode_cols(l4n, p4n):
    lx, ly, lw, lh = l4n[0:1, :], l4n[1:2, :], l4n[2:3, :], l4n[3:4, :]
    px, py, pw, ph = p4n[0:1, :], p4n[1:2, :], p4n[2:3, :], p4n[3:4, :]
    cx = px + lx * VAR0 * pw
    cy = py + ly * VAR0 * ph
    w = pw * jnp.exp(lw * VAR1)
    h = ph * jnp.exp(lh * VAR1)
    x1 = cx - w * 0.5
    y1 = cy - h * 0.5
    x2 = cx + w * 0.5
    y2 = cy + h * 0.5
    area = (x2 - x1) * (y2 - y1)
    return x1, y1, x2, y2, area


def _decode_rows(lr, pr):
    lx, ly, lw, lh = lr[:, 0:1], lr[:, 1:2], lr[:, 2:3], lr[:, 3:4]
    px, py, pw, ph = pr[:, 0:1], pr[:, 1:2], pr[:, 2:3], pr[:, 3:4]
    cx = px + lx * VAR0 * pw
    cy = py + ly * VAR0 * ph
    w = pw * jnp.exp(lw * VAR1)
    h = ph * jnp.exp(lh * VAR1)
    x1 = cx - w * 0.5
    y1 = cy - h * 0.5
    x2 = cx + w * 0.5
    y2 = cy + h * 0.5
    area = (x2 - x1) * (y2 - y1)
    return x1, y1, x2, y2, area


def _rep_kernel(n, g, alist, counts, loc_r, pri_r, m_row, loc_c, pri_c,
                m_col, gt_c, out_ref,
                cx1, cy1, cx2, cy2, cam, accb, accn, accg):
    bi = pl.program_id(0)
    reps = n // 128

    x1c, y1c, x2c, y2c, area_c = _decode_cols(loc_c[0], pri_c[...])
    mc = m_col[0]
    area_cm = jnp.where(mc > 0.0, area_c, NEG)
    cx1[...] = jnp.broadcast_to(x1c, (8, n))
    cy1[...] = jnp.broadcast_to(y1c, (8, n))
    cx2[...] = jnp.broadcast_to(x2c, (8, n))
    cy2[...] = jnp.broadcast_to(y2c, (8, n))
    cam[...] = jnp.broadcast_to(area_cm, (8, n))
    accb[...] = jnp.zeros_like(accb)
    accn[...] = jnp.zeros_like(accn)
    accg[...] = jnp.zeros_like(accg)

    gt = gt_c[0]
    gx1, gy1, gx2, gy2 = gt[0:1, :], gt[1:2, :], gt[2:3, :], gt[3:4, :]
    garea = (gx2 - gx1) * (gy2 - gy1)
    cols = jax.lax.broadcasted_iota(jnp.int32, (8, g), 1)

    def rep(v):
        return pltpu.repeat(jnp.broadcast_to(v, (8, 128)), reps, axis=1)

    def bg(v):
        return jnp.broadcast_to(v, (8, g))

    def body(i, _):
        j = alist[bi, i]
        r0 = pl.multiple_of(j * 8, 8)
        x1r, y1r, x2r, y2r, area_r = _decode_rows(
            loc_r[0, pl.ds(r0, 8), :], pri_r[pl.ds(r0, 8), :])
        mr = m_row[0, pl.ds(r0, 8), :]
        area_rm = jnp.where(mr > 0.0, area_r, NEG)

        iw = jnp.maximum(jnp.minimum(rep(x2r), cx2[...]) -
                         jnp.maximum(rep(x1r), cx1[...]), 0.0)
        ih = jnp.maximum(jnp.minimum(rep(y2r), cy2[...]) -
                         jnp.maximum(rep(y1r), cy1[...]), 0.0)
        inter = iw * ih
        ov = inter / (rep(area_rm) + cam[...] - inter)
        accb[...] += jnp.maximum(ov, 0.0)
        accn[...] += jnp.where(ov > 0.0, 1.0, 0.0)

        giw = jnp.maximum(jnp.minimum(bg(x2r), gx2) -
                          jnp.maximum(bg(x1r), gx1), 0.0)
        gih = jnp.maximum(jnp.minimum(bg(y2r), gy2) -
                          jnp.maximum(bg(y1r), gy1), 0.0)
        ginter = giw * gih
        gov = (ginter / (bg(area_r) + garea - ginter)) * bg(mr)

        max1 = jnp.max(gov, axis=1, keepdims=True)
        arg1 = jnp.min(jnp.where(gov == max1, cols, g), axis=1, keepdims=True)
        ov2 = jnp.where(cols == arg1, 0.0, gov)
        max2 = jnp.max(ov2, axis=1, keepdims=True)
        arg2 = jnp.min(jnp.where(ov2 == max2, cols, g), axis=1, keepdims=True)
        onehot2 = jnp.where(cols == arg2, 1.0, 0.0)

        def sel(v):
            return jnp.sum(onehot2 * v, axis=1, keepdims=True)

        sx1, sy1, sx2, sy2 = sel(gx1), sel(gy1), sel(gx2), sel(gy2)
        sarea = sel(garea)
        iw2 = jnp.maximum(jnp.minimum(x2r, sx2) - jnp.maximum(x1r, sx1), 0.0)
        ih2 = jnp.maximum(jnp.minimum(y2r, sy2) - jnp.maximum(y1r, sy1), 0.0)
        iog = (iw2 * ih2) / sarea
        iog_safe = jnp.where(iog > SIGMA_REPGT, 0.0, iog)
        term = jnp.where(iog > SIGMA_REPGT,
                         (iog - SIGMA_REPGT) / (1.0 - SIGMA_REPGT) - LOG1MS,
                         -jnp.log(jnp.maximum(1.0 - iog_safe, EPS)))
        cf = jnp.where(max2 > 0.0, mr, 0.0)
        accg[:, 0:1] += cf * term
        accg[:, 1:2] += cf
        return 0

    jax.lax.fori_loop(0, counts[bi], body, 0)

    tb = jnp.sum(accb[...])
    nb = jnp.sum(accn[...])
    tg = jnp.sum(accg[:, 0:1])
    ng = jnp.sum(accg[:, 1:2])
    lgt = jnp.where(ng > 0.0, tg / jnp.maximum(ng, 1.0), 0.0)
    lbx = jnp.where(nb > 0.0, tb / jnp.maximum(nb, 1.0), 0.0)
    lane = jax.lax.broadcasted_iota(jnp.int32, (1, 1, 128), 2)
    out_ref[...] = jnp.where(lane == 0, lgt + lbx, 0.0)


@jax.jit
def kernel(loc_data, ground_data, prior_data, pos_idx):
    b, n, _ = loc_data.shape
    g = ground_data.shape[1]
    n_sub = n // 8

    mask = pos_idx[..., 0]
    maskf = mask.astype(jnp.float32)
    loc_col = jnp.transpose(loc_data, (0, 2, 1))
    prior_col = prior_data.T
    gt_col = jnp.transpose(ground_data, (0, 2, 1))
    mask_col = maskf.reshape(b, 1, n)
    mask_row = maskf.reshape(b, n, 1)
    act = jnp.any(mask.reshape(b, n_sub, 8), axis=-1)
    counts = jnp.sum(act, axis=-1).astype(jnp.int32)
    alist = jnp.argsort(~act, axis=-1, stable=True).astype(jnp.int32)

    out = pl.pallas_call(
        functools.partial(_rep_kernel, n, g),
        out_shape=jax.ShapeDtypeStruct((b, 1, 128), jnp.float32),
        grid=(b,),
        in_specs=[
            pl.BlockSpec(memory_space=pltpu.SMEM),
            pl.BlockSpec(memory_space=pltpu.SMEM),
            pl.BlockSpec((1, n, 4), lambda i: (i, 0, 0)),
            pl.BlockSpec((n, 4), lambda i: (0, 0)),
            pl.BlockSpec((1, n, 1), lambda i: (i, 0, 0)),
            pl.BlockSpec((1, 4, n), lambda i: (i, 0, 0)),
            pl.BlockSpec((4, n), lambda i: (0, 0)),
            pl.BlockSpec((1, 1, n), lambda i: (i, 0, 0)),
            pl.BlockSpec((1, 4, g), lambda i: (i, 0, 0)),
        ],
        out_specs=pl.BlockSpec((1, 1, 128), lambda i: (i, 0, 0)),
        scratch_shapes=[
            pltpu.VMEM((8, n), jnp.float32),
            pltpu.VMEM((8, n), jnp.float32),
            pltpu.VMEM((8, n), jnp.float32),
            pltpu.VMEM((8, n), jnp.float32),
            pltpu.VMEM((8, n), jnp.float32),
            pltpu.VMEM((8, n), jnp.float32),
            pltpu.VMEM((8, n), jnp.float32),
            pltpu.VMEM((8, 128), jnp.float32),
        ],
        compiler_params=pltpu.CompilerParams(
            dimension_semantics=("parallel",),
        ),
        name="repulsion_loss",
    )(alist, counts, loc_data, prior_data, mask_row, loc_col, prior_col,
      mask_col, gt_col)

    return jnp.sum(out[:, 0, 0])


# speedup vs baseline: 3.2816x; 2.8810x over previous
import functools

import jax
import jax.numpy as jnp
import numpy as np
from jax.experimental import pallas as pl
from jax.experimental.pallas import tpu as pltpu

VAR0 = 0.1
VAR1 = 0.2
SIGMA_REPGT = 0.9
EPS = 1e-10
LOG1MS = np.float32(np.log(1.0 - SIGMA_REPGT))
NEG = np.float32(-np.inf)


def _decode_cols(l4n, p4n):
    lx, ly, lw, lh = l4n[0:1, :], l4n[1:2, :], l4n[2:3, :], l4n[3:4, :]
    px, py, pw, ph = p4n[0:1, :], p4n[1:2, :], p4n[2:3, :], p4n[3:4, :]
    cx = px + lx * VAR0 * pw
    cy = py + ly * VAR0 * ph
    w = pw * jnp.exp(lw * VAR1)
    h = ph * jnp.exp(lh * VAR1)
    x1 = cx - w * 0.5
    y1 = cy - h * 0.5
    x2 = cx + w * 0.5
    y2 = cy + h * 0.5
    area = (x2 - x1) * (y2 - y1)
    return x1, y1, x2, y2, area


def _decode_rows(lr, pr):
    lx, ly, lw, lh = lr[:, 0:1], lr[:, 1:2], lr[:, 2:3], lr[:, 3:4]
    px, py, pw, ph = pr[:, 0:1], pr[:, 1:2], pr[:, 2:3], pr[:, 3:4]
    cx = px + lx * VAR0 * pw
    cy = py + ly * VAR0 * ph
    w = pw * jnp.exp(lw * VAR1)
    h = ph * jnp.exp(lh * VAR1)
    x1 = cx - w * 0.5
    y1 = cy - h * 0.5
    x2 = cx + w * 0.5
    y2 = cy + h * 0.5
    area = (x2 - x1) * (y2 - y1)
    return x1, y1, x2, y2, area


def _rep_kernel(n, g, alist, counts, loc_r, pri_r, m_row, loc_c, pri_c,
                m_col, gt_r, out_ref,
                cx1, cy1, cx2, cy2, cam, cat, cmsk, govs, accb, accn):
    bi = pl.program_id(0)
    reps = n // 128
    n_grp = g // 8

    def rep(v):
        return pltpu.repeat(jnp.broadcast_to(v, (8, 128)), reps, axis=1)

    x1c, y1c, x2c, y2c, area_c = _decode_cols(loc_c[0], pri_c[...])
    mc = m_col[0]
    area_cm = jnp.where(mc > 0.0, area_c, NEG)
    cx1[...] = jnp.broadcast_to(x1c, (8, n))
    cy1[...] = jnp.broadcast_to(y1c, (8, n))
    cx2[...] = jnp.broadcast_to(x2c, (8, n))
    cy2[...] = jnp.broadcast_to(y2c, (8, n))
    cam[...] = jnp.broadcast_to(area_cm, (8, n))
    cat[...] = jnp.broadcast_to(area_c, (8, n))
    cmsk[...] = jnp.broadcast_to(mc, (8, n))
    accb[...] = jnp.zeros_like(accb)
    accn[...] = jnp.zeros_like(accn)

    gr = gt_r[0]
    si8 = jax.lax.broadcasted_iota(jnp.int32, (8, n), 0)

    def gt_group(gg):
        g4 = gr[gg * 8:(gg + 1) * 8, :]
        gx1, gy1 = g4[:, 0:1], g4[:, 1:2]
        gx2, gy2 = g4[:, 2:3], g4[:, 3:4]
        garea = (gx2 - gx1) * (gy2 - gy1)
        return gx1, gy1, gx2, gy2, garea

    def inter_group(gg):
        gx1, gy1, gx2, gy2, garea = gt_group(gg)
        iw = jnp.maximum(jnp.minimum(rep(gx2), cx2[...]) -
                         jnp.maximum(rep(gx1), cx1[...]), 0.0)
        ih = jnp.maximum(jnp.minimum(rep(gy2), cy2[...]) -
                         jnp.maximum(rep(gy1), cy1[...]), 0.0)
        return iw * ih, garea

    m1 = None
    for gg in range(n_grp):
        inter, garea = inter_group(gg)
        iou = inter / (rep(garea) + cat[...] - inter)
        gov = iou * cmsk[...]
        govs[gg * 8:(gg + 1) * 8, :] = gov
        gm = jnp.max(gov, axis=0, keepdims=True)
        gi = jnp.min(jnp.where(gov == gm, si8, 127), axis=0,
                     keepdims=True) + gg * 8
        if m1 is None:
            m1, a1 = gm, gi
        else:
            take = gm > m1
            m1 = jnp.maximum(m1, gm)
            a1 = jnp.where(take, gi, a1)

    m2 = None
    for gg in range(n_grp):
        sidx = si8 + gg * 8
        ov2 = jnp.where(sidx == a1, 0.0, govs[gg * 8:(gg + 1) * 8, :])
        gm = jnp.max(ov2, axis=0, keepdims=True)
        gi = jnp.min(jnp.where(ov2 == gm, si8, 127), axis=0,
                     keepdims=True) + gg * 8
        if m2 is None:
            m2, a2 = gm, gi
        else:
            take = gm > m2
            m2 = jnp.maximum(m2, gm)
            a2 = jnp.where(take, gi, a2)

    iog_acc = jnp.zeros((8, n), jnp.float32)
    for gg in range(n_grp):
        inter, garea = inter_group(gg)
        iog_g = inter * rep(1.0 / garea)
        iog_acc = iog_acc + jnp.where(si8 + gg * 8 == a2, iog_g, 0.0)
    iog = jnp.sum(iog_acc, axis=0, keepdims=True)

    iog_safe = jnp.where(iog > SIGMA_REPGT, 0.0, iog)
    term = jnp.where(iog > SIGMA_REPGT,
                     (iog - SIGMA_REPGT) / (1.0 - SIGMA_REPGT) - LOG1MS,
                     -jnp.log(jnp.maximum(1.0 - iog_safe, EPS)))
    cf = jnp.where(m2 > 0.0, mc, 0.0)
    tg = jnp.sum(cf * term)
    ng = jnp.sum(cf)

    def body(i, _):
        j = alist[bi, i]
        r0 = pl.multiple_of(j * 8, 8)
        x1r, y1r, x2r, y2r, area_r = _decode_rows(
            loc_r[0, pl.ds(r0, 8), :], pri_r[pl.ds(r0, 8), :])
        mr = m_row[0, pl.ds(r0, 8), :]
        area_rm = jnp.where(mr > 0.0, area_r, NEG)

        iw = jnp.maximum(jnp.minimum(rep(x2r), cx2[...]) -
                         jnp.maximum(rep(x1r), cx1[...]), 0.0)
        ih = jnp.maximum(jnp.minimum(rep(y2r), cy2[...]) -
                         jnp.maximum(rep(y1r), cy1[...]), 0.0)
        inter = iw * ih
        ov = inter / (rep(area_rm) + cam[...] - inter)
        accb[...] += jnp.maximum(ov, 0.0)
        accn[...] += jnp.where(ov > 0.0, 1.0, 0.0)
        return 0

    jax.lax.fori_loop(0, counts[bi], body, 0)

    tb = jnp.sum(accb[...])
    nb = jnp.sum(accn[...])
    lgt = jnp.where(ng > 0.0, tg / jnp.maximum(ng, 1.0), 0.0)
    lbx = jnp.where(nb > 0.0, tb / jnp.maximum(nb, 1.0), 0.0)
    lane = jax.lax.broadcasted_iota(jnp.int32, (1, 1, 128), 2)
    out_ref[...] = jnp.where(lane == 0, lgt + lbx, 0.0)


@jax.jit
def kernel(loc_data, ground_data, prior_data, pos_idx):
    b, n, _ = loc_data.shape
    g = ground_data.shape[1]
    n_sub = n // 8

    mask = pos_idx[..., 0]
    maskf = mask.astype(jnp.float32)
    loc_col = jnp.transpose(loc_data, (0, 2, 1))
    prior_col = prior_data.T
    mask_col = maskf.reshape(b, 1, n)
    mask_row = maskf.reshape(b, n, 1)
    act = jnp.any(mask.reshape(b, n_sub, 8), axis=-1)
    counts = jnp.sum(act, axis=-1).astype(jnp.int32)
    alist = jnp.argsort(~act, axis=-1, stable=True).astype(jnp.int32)

    out = pl.pallas_call(
        functools.partial(_rep_kernel, n, g),
        out_shape=jax.ShapeDtypeStruct((b, 1, 128), jnp.float32),
        grid=(b,),
        in_specs=[
            pl.BlockSpec(memory_space=pltpu.SMEM),
            pl.BlockSpec(memory_space=pltpu.SMEM),
            pl.BlockSpec((1, n, 4), lambda i: (i, 0, 0)),
            pl.BlockSpec((n, 4), lambda i: (0, 0)),
            pl.BlockSpec((1, n, 1), lambda i: (i, 0, 0)),
            pl.BlockSpec((1, 4, n), lambda i: (i, 0, 0)),
            pl.BlockSpec((4, n), lambda i: (0, 0)),
            pl.BlockSpec((1, 1, n), lambda i: (i, 0, 0)),
            pl.BlockSpec((1, g, 4), lambda i: (i, 0, 0)),
        ],
        out_specs=pl.BlockSpec((1, 1, 128), lambda i: (i, 0, 0)),
        scratch_shapes=[
            pltpu.VMEM((8, n), jnp.float32),
            pltpu.VMEM((8, n), jnp.float32),
            pltpu.VMEM((8, n), jnp.float32),
            pltpu.VMEM((8, n), jnp.float32),
            pltpu.VMEM((8, n), jnp.float32),
            pltpu.VMEM((8, n), jnp.float32),
            pltpu.VMEM((8, n), jnp.float32),
            pltpu.VMEM((g, n), jnp.float32),
            pltpu.VMEM((8, n), jnp.float32),
            pltpu.VMEM((8, n), jnp.float32),
        ],
        compiler_params=pltpu.CompilerParams(
            dimension_semantics=("parallel",),
        ),
        name="repulsion_loss",
    )(alist, counts, loc_data, prior_data, mask_row, loc_col, prior_col,
      mask_col, ground_data)

    return jnp.sum(out[:, 0, 0])


# speedup vs baseline: 9.4821x; 2.8895x over previous
import functools

import jax
import jax.numpy as jnp
import numpy as np
from jax.experimental import pallas as pl
from jax.experimental.pallas import tpu as pltpu

VAR0 = 0.1
VAR1 = 0.2
SIGMA_REPGT = 0.9
EPS = 1e-10
LOG1MS = np.float32(np.log(1.0 - SIGMA_REPGT))
NEG = np.float32(-np.inf)


def _decode_cols(l4n, p4n):
    lx, ly, lw, lh = l4n[0:1, :], l4n[1:2, :], l4n[2:3, :], l4n[3:4, :]
    px, py, pw, ph = p4n[0:1, :], p4n[1:2, :], p4n[2:3, :], p4n[3:4, :]
    cx = px + lx * VAR0 * pw
    cy = py + ly * VAR0 * ph
    w = pw * jnp.exp(lw * VAR1)
    h = ph * jnp.exp(lh * VAR1)
    x1 = cx - w * 0.5
    y1 = cy - h * 0.5
    x2 = cx + w * 0.5
    y2 = cy + h * 0.5
    area = (x2 - x1) * (y2 - y1)
    return x1, y1, x2, y2, area


def _rep_kernel(n, g, alist, counts, loc_c, pri_c, m_col, gt_r, out_ref,
                cx1, cy1, cx2, cy2, cam, cat, cmsk, govs, accb, accn,
                rx1, ry1, rx2, ry2, ram):
    bi = pl.program_id(0)
    reps = n // 128
    n_grp = g // 8

    def rep(v):
        return pltpu.repeat(jnp.broadcast_to(v, (8, 128)), reps, axis=1)

    x1c, y1c, x2c, y2c, area_c = _decode_cols(loc_c[0], pri_c[...])
    mc = m_col[0]
    area_cm = jnp.where(mc > 0.0, area_c, NEG)
    cx1[...] = jnp.broadcast_to(x1c, (8, n))
    cy1[...] = jnp.broadcast_to(y1c, (8, n))
    cx2[...] = jnp.broadcast_to(x2c, (8, n))
    cy2[...] = jnp.broadcast_to(y2c, (8, n))
    cam[...] = jnp.broadcast_to(area_cm, (8, n))
    cat[...] = jnp.broadcast_to(area_c, (8, n))
    cmsk[...] = jnp.broadcast_to(mc, (8, n))
    accb[...] = jnp.zeros_like(accb)
    accn[...] = jnp.zeros_like(accn)

    for k in range(reps):
        sl = slice(k * 128, (k + 1) * 128)
        for dst, src in ((rx1, x1c), (ry1, y1c), (rx2, x2c), (ry2, y2c),
                         (ram, area_cm)):
            dst[sl, :] = jnp.broadcast_to(src[0:1, sl], (128, 128)).T

    gr = gt_r[0]
    si8 = jax.lax.broadcasted_iota(jnp.int32, (8, n), 0)

    def gt_group(gg):
        g4 = gr[gg * 8:(gg + 1) * 8, :]
        gx1, gy1 = g4[:, 0:1], g4[:, 1:2]
        gx2, gy2 = g4[:, 2:3], g4[:, 3:4]
        garea = (gx2 - gx1) * (gy2 - gy1)
        return gx1, gy1, gx2, gy2, garea

    def inter_group(gg):
        gx1, gy1, gx2, gy2, garea = gt_group(gg)
        iw = jnp.maximum(jnp.minimum(rep(gx2), cx2[...]) -
                         jnp.maximum(rep(gx1), cx1[...]), 0.0)
        ih = jnp.maximum(jnp.minimum(rep(gy2), cy2[...]) -
                         jnp.maximum(rep(gy1), cy1[...]), 0.0)
        return iw * ih, garea

    m1 = None
    for gg in range(n_grp):
        inter, garea = inter_group(gg)
        iou = inter / (rep(garea) + cat[...] - inter)
        gov = iou * cmsk[...]
        govs[gg * 8:(gg + 1) * 8, :] = gov
        gm = jnp.max(gov, axis=0, keepdims=True)
        gi = jnp.min(jnp.where(gov == gm, si8, 127), axis=0,
                     keepdims=True) + gg * 8
        if m1 is None:
            m1, a1 = gm, gi
        else:
            take = gm > m1
            m1 = jnp.maximum(m1, gm)
            a1 = jnp.where(take, gi, a1)

    m2 = None
    for gg in range(n_grp):
        sidx = si8 + gg * 8
        ov2 = jnp.where(sidx == a1, 0.0, govs[gg * 8:(gg + 1) * 8, :])
        gm = jnp.max(ov2, axis=0, keepdims=True)
        gi = jnp.min(jnp.where(ov2 == gm, si8, 127), axis=0,
                     keepdims=True) + gg * 8
        if m2 is None:
            m2, a2 = gm, gi
        else:
            take = gm > m2
            m2 = jnp.maximum(m2, gm)
            a2 = jnp.where(take, gi, a2)

    iog_acc = jnp.zeros((8, n), jnp.float32)
    for gg in range(n_grp):
        inter, garea = inter_group(gg)
        iog_g = inter * rep(1.0 / garea)
        iog_acc = iog_acc + jnp.where(si8 + gg * 8 == a2, iog_g, 0.0)
    iog = jnp.sum(iog_acc, axis=0, keepdims=True)

    iog_safe = jnp.where(iog > SIGMA_REPGT, 0.0, iog)
    term = jnp.where(iog > SIGMA_REPGT,
                     (iog - SIGMA_REPGT) / (1.0 - SIGMA_REPGT) - LOG1MS,
                     -jnp.log(jnp.maximum(1.0 - iog_safe, EPS)))
    cf = jnp.where(m2 > 0.0, mc, 0.0)
    tg = jnp.sum(cf * term)
    ng = jnp.sum(cf)

    def repv(v):
        return pltpu.repeat(v, reps, axis=1)

    def body(i, _):
        for u in range(2):
            j = alist[bi, i * 2 + u]
            r0 = pl.multiple_of(j * 8, 8)
            x1r = rx1[pl.ds(r0, 8), :]
            y1r = ry1[pl.ds(r0, 8), :]
            x2r = rx2[pl.ds(r0, 8), :]
            y2r = ry2[pl.ds(r0, 8), :]
            armr = ram[pl.ds(r0, 8), :]

            iw = jnp.maximum(jnp.minimum(repv(x2r), cx2[...]) -
                             jnp.maximum(repv(x1r), cx1[...]), 0.0)
            ih = jnp.maximum(jnp.minimum(repv(y2r), cy2[...]) -
                             jnp.maximum(repv(y1r), cy1[...]), 0.0)
            inter = iw * ih
            ov = inter / (repv(armr) + cam[...] - inter)
            ovp = jnp.maximum(ov, 0.0)
            accb[...] += ovp
            accn[...] += jnp.where(ovp > 0.0, 1.0, 0.0)
        return 0

    jax.lax.fori_loop(0, (counts[bi] + 1) // 2, body, 0)

    tb = jnp.sum(accb[...])
    nb = jnp.sum(accn[...])
    lgt = jnp.where(ng > 0.0, tg / jnp.maximum(ng, 1.0), 0.0)
    lbx = jnp.where(nb > 0.0, tb / jnp.maximum(nb, 1.0), 0.0)
    lane = jax.lax.broadcasted_iota(jnp.int32, (1, 1, 128), 2)
    out_ref[...] = jnp.where(lane == 0, lgt + lbx, 0.0)


@jax.jit
def kernel(loc_data, ground_data, prior_data, pos_idx):
    b, n, _ = loc_data.shape
    g = ground_data.shape[1]
    n_sub = n // 8

    mask = pos_idx[..., 0]
    maskf = mask.astype(jnp.float32)
    loc_col = jnp.transpose(loc_data, (0, 2, 1))
    prior_col = prior_data.T
    mask_col = maskf.reshape(b, 1, n)
    act = jnp.any(mask.reshape(b, n_sub, 8), axis=-1)
    counts = jnp.sum(act, axis=-1).astype(jnp.int32)
    alist = jnp.argsort(~act, axis=-1, stable=True).astype(jnp.int32)

    out = pl.pallas_call(
        functools.partial(_rep_kernel, n, g),
        out_shape=jax.ShapeDtypeStruct((b, 1, 128), jnp.float32),
        grid=(b,),
        in_specs=[
            pl.BlockSpec(memory_space=pltpu.SMEM),
            pl.BlockSpec(memory_space=pltpu.SMEM),
            pl.BlockSpec((1, 4, n), lambda i: (i, 0, 0)),
            pl.BlockSpec((4, n), lambda i: (0, 0)),
            pl.BlockSpec((1, 1, n), lambda i: (i, 0, 0)),
            pl.BlockSpec((1, g, 4), lambda i: (i, 0, 0)),
        ],
        out_specs=pl.BlockSpec((1, 1, 128), lambda i: (i, 0, 0)),
        scratch_shapes=[
            pltpu.VMEM((8, n), jnp.float32),
            pltpu.VMEM((8, n), jnp.float32),
            pltpu.VMEM((8, n), jnp.float32),
            pltpu.VMEM((8, n), jnp.float32),
            pltpu.VMEM((8, n), jnp.float32),
            pltpu.VMEM((8, n), jnp.float32),
            pltpu.VMEM((8, n), jnp.float32),
            pltpu.VMEM((g, n), jnp.float32),
            pltpu.VMEM((8, n), jnp.float32),
            pltpu.VMEM((8, n), jnp.float32),
            pltpu.VMEM((n, 128), jnp.float32),
            pltpu.VMEM((n, 128), jnp.float32),
            pltpu.VMEM((n, 128), jnp.float32),
            pltpu.VMEM((n, 128), jnp.float32),
            pltpu.VMEM((n, 128), jnp.float32),
        ],
        compiler_params=pltpu.CompilerParams(
            dimension_semantics=("parallel",),
        ),
        name="repulsion_loss",
    )(alist, counts, loc_col, prior_col, mask_col, ground_data)

    return jnp.sum(out[:, 0, 0])


# speedup vs baseline: 9.4949x; 1.0014x over previous
import functools

import jax
import jax.numpy as jnp
import numpy as np
from jax.experimental import pallas as pl
from jax.experimental.pallas import tpu as pltpu

VAR0 = 0.1
VAR1 = 0.2
SIGMA_REPGT = 0.9
EPS = 1e-10
LOG1MS = np.float32(np.log(1.0 - SIGMA_REPGT))
NEG = np.float32(-np.inf)


def _decode_cols(l4n, p4n):
    lx, ly, lw, lh = l4n[0:1, :], l4n[1:2, :], l4n[2:3, :], l4n[3:4, :]
    px, py, pw, ph = p4n[0:1, :], p4n[1:2, :], p4n[2:3, :], p4n[3:4, :]
    cx = px + lx * VAR0 * pw
    cy = py + ly * VAR0 * ph
    w = pw * jnp.exp(lw * VAR1)
    h = ph * jnp.exp(lh * VAR1)
    x1 = cx - w * 0.5
    y1 = cy - h * 0.5
    x2 = cx + w * 0.5
    y2 = cy + h * 0.5
    area = (x2 - x1) * (y2 - y1)
    return x1, y1, x2, y2, area


def _rep_kernel(n, g, alist, counts, loc_c, pri_c, m_col, gt_r, out_ref,
                cx1, cy1, cx2, cy2, cam, cat, cmsk, govs, accb, accn,
                rx1, ry1, rx2, ry2, ram):
    bi = pl.program_id(0)
    reps = n // 128
    n_grp = g // 8

    def rep(v):
        return pltpu.repeat(jnp.broadcast_to(v, (8, 128)), reps, axis=1)

    x1c, y1c, x2c, y2c, area_c = _decode_cols(loc_c[0], pri_c[...])
    mc = m_col[0]
    area_cm = jnp.where(mc > 0.0, area_c, NEG)
    cx1[...] = jnp.broadcast_to(x1c, (8, n))
    cy1[...] = jnp.broadcast_to(y1c, (8, n))
    cx2[...] = jnp.broadcast_to(x2c, (8, n))
    cy2[...] = jnp.broadcast_to(y2c, (8, n))
    cam[...] = jnp.broadcast_to(area_cm, (8, n))
    cat[...] = jnp.broadcast_to(area_c, (8, n))
    cmsk[...] = jnp.broadcast_to(mc, (8, n))
    accb[...] = jnp.zeros_like(accb)
    accn[...] = jnp.zeros_like(accn)

    for k in range(reps):
        sl = slice(k * 128, (k + 1) * 128)
        for dst, src in ((rx1, x1c), (ry1, y1c), (rx2, x2c), (ry2, y2c),
                         (ram, area_cm)):
            dst[sl, :] = jnp.broadcast_to(src[0:1, sl], (128, 128)).T

    gr = gt_r[0]
    si8 = jax.lax.broadcasted_iota(jnp.int32, (8, n), 0)

    def gt_group(gg):
        g4 = gr[gg * 8:(gg + 1) * 8, :]
        gx1, gy1 = g4[:, 0:1], g4[:, 1:2]
        gx2, gy2 = g4[:, 2:3], g4[:, 3:4]
        garea = (gx2 - gx1) * (gy2 - gy1)
        return gx1, gy1, gx2, gy2, garea

    def inter_group(gg):
        gx1, gy1, gx2, gy2, garea = gt_group(gg)
        iw = jnp.maximum(jnp.minimum(rep(gx2), cx2[...]) -
                         jnp.maximum(rep(gx1), cx1[...]), 0.0)
        ih = jnp.maximum(jnp.minimum(rep(gy2), cy2[...]) -
                         jnp.maximum(rep(gy1), cy1[...]), 0.0)
        return iw * ih, garea

    m1 = None
    for gg in range(n_grp):
        inter, garea = inter_group(gg)
        iou = inter / (rep(garea) + cat[...] - inter)
        gov = iou * cmsk[...]
        govs[gg * 8:(gg + 1) * 8, :] = gov
        gm = jnp.max(gov, axis=0, keepdims=True)
        gi = jnp.min(jnp.where(gov == gm, si8, 127), axis=0,
                     keepdims=True) + gg * 8
        if m1 is None:
            m1, a1 = gm, gi
        else:
            take = gm > m1
            m1 = jnp.maximum(m1, gm)
            a1 = jnp.where(take, gi, a1)

    m2 = None
    for gg in range(n_grp):
        sidx = si8 + gg * 8
        ov2 = jnp.where(sidx == a1, 0.0, govs[gg * 8:(gg + 1) * 8, :])
        gm = jnp.max(ov2, axis=0, keepdims=True)
        gi = jnp.min(jnp.where(ov2 == gm, si8, 127), axis=0,
                     keepdims=True) + gg * 8
        if m2 is None:
            m2, a2 = gm, gi
        else:
            take = gm > m2
            m2 = jnp.maximum(m2, gm)
            a2 = jnp.where(take, gi, a2)

    iog_acc = jnp.zeros((8, n), jnp.float32)
    for gg in range(n_grp):
        inter, garea = inter_group(gg)
        iog_g = inter * rep(1.0 / garea)
        iog_acc = iog_acc + jnp.where(si8 + gg * 8 == a2, iog_g, 0.0)
    iog = jnp.sum(iog_acc, axis=0, keepdims=True)

    iog_safe = jnp.where(iog > SIGMA_REPGT, 0.0, iog)
    term = jnp.where(iog > SIGMA_REPGT,
                     (iog - SIGMA_REPGT) / (1.0 - SIGMA_REPGT) - LOG1MS,
                     -jnp.log(jnp.maximum(1.0 - iog_safe, EPS)))
    cf = jnp.where(m2 > 0.0, mc, 0.0)
    tg = jnp.sum(cf * term)
    ng = jnp.sum(cf)

    def repv(v):
        return pltpu.repeat(v, reps, axis=1)

    def body(i, _):
        for u in range(2):
            j = alist[bi, i * 2 + u]
            r0 = pl.multiple_of(j * 8, 8)
            x1r = rx1[pl.ds(r0, 8), :]
            y1r = ry1[pl.ds(r0, 8), :]
            x2r = rx2[pl.ds(r0, 8), :]
            y2r = ry2[pl.ds(r0, 8), :]
            armr = ram[pl.ds(r0, 8), :]

            iw = jnp.maximum(jnp.minimum(repv(x2r), cx2[...]) -
                             jnp.maximum(repv(x1r), cx1[...]), 0.0)
            ih = jnp.maximum(jnp.minimum(repv(y2r), cy2[...]) -
                             jnp.maximum(repv(y1r), cy1[...]), 0.0)
            inter = iw * ih
            ov = inter / (repv(armr) + cam[...] - inter)
            ovp = jnp.maximum(ov, 0.0)
            accb[...] += ovp
            accn[...] += jnp.where(ovp > 0.0, 1.0, 0.0)
        return 0

    jax.lax.fori_loop(0, (counts[bi] + 1) // 2, body, 0)

    tb = jnp.sum(accb[...])
    nb = jnp.sum(accn[...])
    lgt = jnp.where(ng > 0.0, tg / jnp.maximum(ng, 1.0), 0.0)
    lbx = jnp.where(nb > 0.0, tb / jnp.maximum(nb, 1.0), 0.0)
    lane = jax.lax.broadcasted_iota(jnp.int32, (1, 1, 128), 2)
    out_ref[...] = jnp.where(lane == 0, lgt + lbx, 0.0)


@jax.jit
def kernel(loc_data, ground_data, prior_data, pos_idx):
    b, n, _ = loc_data.shape
    g = ground_data.shape[1]
    n_sub = n // 8

    mask = pos_idx[..., 0]
    maskf = mask.astype(jnp.float32)
    loc_col = jnp.transpose(loc_data, (0, 2, 1))
    prior_col = prior_data.T
    mask_col = maskf.reshape(b, 1, n)
    act = jnp.any(mask.reshape(b, n_sub, 8), axis=-1)
    counts = jnp.sum(act, axis=-1).astype(jnp.int32)
    alist = jnp.argsort(~act, axis=-1, stable=False).astype(jnp.int32)

    out = pl.pallas_call(
        functools.partial(_rep_kernel, n, g),
        out_shape=jax.ShapeDtypeStruct((b, 1, 128), jnp.float32),
        grid=(b,),
        in_specs=[
            pl.BlockSpec(memory_space=pltpu.SMEM),
            pl.BlockSpec(memory_space=pltpu.SMEM),
            pl.BlockSpec((1, 4, n), lambda i: (i, 0, 0)),
            pl.BlockSpec((4, n), lambda i: (0, 0)),
            pl.BlockSpec((1, 1, n), lambda i: (i, 0, 0)),
            pl.BlockSpec((1, g, 4), lambda i: (i, 0, 0)),
        ],
        out_specs=pl.BlockSpec((1, 1, 128), lambda i: (i, 0, 0)),
        scratch_shapes=[
            pltpu.VMEM((8, n), jnp.float32),
            pltpu.VMEM((8, n), jnp.float32),
            pltpu.VMEM((8, n), jnp.float32),
            pltpu.VMEM((8, n), jnp.float32),
            pltpu.VMEM((8, n), jnp.float32),
            pltpu.VMEM((8, n), jnp.float32),
            pltpu.VMEM((8, n), jnp.float32),
            pltpu.VMEM((g, n), jnp.float32),
            pltpu.VMEM((8, n), jnp.float32),
            pltpu.VMEM((8, n), jnp.float32),
            pltpu.VMEM((n, 128), jnp.float32),
            pltpu.VMEM((n, 128), jnp.float32),
            pltpu.VMEM((n, 128), jnp.float32),
            pltpu.VMEM((n, 128), jnp.float32),
            pltpu.VMEM((n, 128), jnp.float32),
        ],
        compiler_params=pltpu.CompilerParams(
            dimension_semantics=("parallel",),
        ),
        name="repulsion_loss",
    )(alist, counts, loc_col, prior_col, mask_col, ground_data)

    return jnp.sum(out[:, 0, 0])
